# baseline (speedup 1.0000x reference)
import functools

import jax
import jax.numpy as jnp
from jax import lax
from jax.experimental import pallas as pl
from jax.experimental.pallas import tpu as pltpu
from jax.experimental.pallas import tpu_sc as plsc

_NUM_LAYER = 3
_EMB = 128
_DE = 16
_N = 10000
_E = 320000
_TASKS = 10
_G = 128

_NC = 2
_NS = 16
_NW = _NC * _NS
_EPT = _E // _NW
_CH = 80
_NCH = _EPT // _CH
_RA = 632
_RT0 = _RA * (_NS - 1)
_RT = _N - _RT0

_MLP_B = 1000


def _sc_mesh():
    return plsc.VectorSubcoreMesh(
        core_axis_name="c", subcore_axis_name="s",
        num_cores=_NC, num_subcores=_NS)


_NBUF = 3
_NBUFG = 4


def _sc_gather_scatter_body(h_hbm, src_hbm, dst_hbm, zero_hbm, out_hbm,
                            srcb0, srcb1, srcb2, srcb3,
                            dstb0, dstb1, dstb2, dstb3,
                            rows0, rows1, rows2, rows3,
                            acc_sh, sem0, sem1, sem2, sem3):
    c = lax.axis_index("c")
    s = lax.axis_index("s")
    srcb = [srcb0, srcb1, srcb2, srcb3]
    dstb = [dstb0, dstb1, dstb2, dstb3]
    rows = [rows0, rows1, rows2, rows3]
    sems = [sem0, sem1, sem2, sem3]
    wid = c * _NS + s

    @pl.when(s < _NS - 1)
    def _():
        pltpu.sync_copy(zero_hbm.at[pl.ds(s * _RA, _RA)],
                        acc_sh.at[pl.ds(s * _RA, _RA)])

    @pl.when(s == _NS - 1)
    def _():
        pltpu.sync_copy(zero_hbm.at[pl.ds(_RT0, _RT)],
                        acc_sh.at[pl.ds(_RT0, _RT)])

    plsc.subcore_barrier()

    base = wid * _EPT

    def gather(i, j):
        pltpu.sync_copy(src_hbm.at[pl.ds(base + i * _CH, _CH)], srcb[j])
        pltpu.async_copy(h_hbm.at[srcb[j]], rows[j], sems[j])
        pltpu.sync_copy(dst_hbm.at[pl.ds(base + i * _CH, _CH)], dstb[j])

    def drain(i, j):
        pltpu.make_async_copy(h_hbm.at[srcb[j]], rows[j], sems[j]).wait()
        pltpu.sync_copy(rows[j], acc_sh.at[dstb[j]], add=True)

    for j in range(_NBUFG - 1):
        gather(j, j)

    def group(q, carry):
        i0 = q * _NBUFG
        for j in range(_NBUFG):
            drain(i0 + j, j)
            gather(i0 + j + (_NBUFG - 1), (j + _NBUFG - 1) % _NBUFG)
        return carry

    nq = (_NCH - _NBUFG + 1) // _NBUFG
    lax.fori_loop(0, nq, group, 0)

    for i in range(nq * _NBUFG, _NCH):
        drain(i, i % _NBUFG)
        nxt = i + _NBUFG - 1
        if nxt < _NCH:
            gather(nxt, nxt % _NBUFG)

    plsc.subcore_barrier()

    @pl.when(s < _NS - 1)
    def _():
        pltpu.sync_copy(acc_sh.at[pl.ds(s * _RA, _RA)],
                        out_hbm.at[c, pl.ds(s * _RA, _RA)])

    @pl.when(s == _NS - 1)
    def _():
        pltpu.sync_copy(acc_sh.at[pl.ds(_RT0, _RT)],
                        out_hbm.at[c, pl.ds(_RT0, _RT)])


@functools.lru_cache(maxsize=None)
def _sc_gather_scatter_kernel():
    return pl.kernel(
        _sc_gather_scatter_body,
        out_type=jax.ShapeDtypeStruct((_NC, _N, _EMB), jnp.float32),
        mesh=_sc_mesh(),
        scratch_types=(
            [pltpu.VMEM((_CH,), jnp.int32)] * 4
            + [pltpu.VMEM((_CH,), jnp.int32)] * 4
            + [pltpu.VMEM((_CH, _EMB), jnp.float32)] * 4
            + [pltpu.VMEM_SHARED((_N, _EMB), jnp.float32)]
            + [pltpu.SemaphoreType.DMA] * 4
        ),
    )


def _sc_gather_scatter(h, src, dst, zeros_big):
    return _sc_gather_scatter_kernel()(h, src, dst, zeros_big)


def _sc_linear_scatter_body(rows_hbm, dst3_hbm, zero_hbm, out_hbm,
                            dst_all, rows0, rows1, rows2,
                            acc_sh, sem0, sem1, sem2):
    c = lax.axis_index("c")
    s = lax.axis_index("s")
    rows = [rows0, rows1, rows2]
    sems = [sem0, sem1, sem2]
    wid = c * _NS + s

    @pl.when(s < _NS - 1)
    def _():
        pltpu.sync_copy(zero_hbm.at[pl.ds(s * _RA, _RA)],
                        acc_sh.at[pl.ds(s * _RA, _RA)])

    @pl.when(s == _NS - 1)
    def _():
        pltpu.sync_copy(zero_hbm.at[pl.ds(_RT0, _RT)],
                        acc_sh.at[pl.ds(_RT0, _RT)])

    pltpu.sync_copy(dst3_hbm.at[wid], dst_all)
    plsc.subcore_barrier()

    base = wid * _EPT

    def gather(i, j):
        pltpu.async_copy(rows_hbm.at[pl.ds(base + i * _CH, _CH)],
                         rows[j], sems[j])

    def drain(i, j):
        pltpu.make_async_copy(rows_hbm.at[pl.ds(base, _CH)],
                              rows[j], sems[j]).wait()
        pltpu.sync_copy(rows[j], acc_sh.at[dst_all.at[i]], add=True)

    for j in range(_NBUF - 1):
        gather(j, j)

    def group(q, carry):
        i0 = q * _NBUF
        for j in range(_NBUF):
            drain(i0 + j, j)
            gather(i0 + j + (_NBUF - 1), (j + _NBUF - 1) % _NBUF)
        return carry

    nq = (_NCH - _NBUF + 1) // _NBUF
    lax.fori_loop(0, nq, group, 0)

    for i in range(nq * _NBUF, _NCH):
        drain(i, i % _NBUF)
        nxt = i + _NBUF - 1
        if nxt < _NCH:
            gather(nxt, nxt % _NBUF)

    plsc.subcore_barrier()

    @pl.when(s < _NS - 1)
    def _():
        pltpu.sync_copy(acc_sh.at[pl.ds(s * _RA, _RA)],
                        out_hbm.at[c, pl.ds(s * _RA, _RA)])

    @pl.when(s == _NS - 1)
    def _():
        pltpu.sync_copy(acc_sh.at[pl.ds(_RT0, _RT)],
                        out_hbm.at[c, pl.ds(_RT0, _RT)])


@functools.lru_cache(maxsize=None)
def _sc_linear_scatter_kernel():
    return pl.kernel(
        _sc_linear_scatter_body,
        out_type=jax.ShapeDtypeStruct((_NC, _N, _EMB), jnp.float32),
        mesh=_sc_mesh(),
        scratch_types=[
            pltpu.VMEM((_NCH, _CH), jnp.int32),
            pltpu.VMEM((_CH, _EMB), jnp.float32),
            pltpu.VMEM((_CH, _EMB), jnp.float32),
            pltpu.VMEM((_CH, _EMB), jnp.float32),
            pltpu.VMEM_SHARED((_N, _EMB), jnp.float32),
            pltpu.SemaphoreType.DMA,
            pltpu.SemaphoreType.DMA,
            pltpu.SemaphoreType.DMA,
        ],
    )


def _sc_edge_sums(ea_aug, dst3, zeros_big):
    return _sc_linear_scatter_kernel()(ea_aug, dst3, zeros_big)


def _mlp_body(last, p_ref, a_ref, h_ref, weh_ref, wee_ref, be_ref,
              w1_ref, b1_ref, w2_ref, b2_ref, o_ref):
    p = p_ref[0] + p_ref[1]
    a = a_ref[0] + a_ref[1]
    e = a[:, :_DE]
    deg = a[:, _DE:_DE + 1]
    z = jnp.dot(p, weh_ref[...], preferred_element_type=jnp.float32, precision=lax.Precision.HIGHEST)
    z = z + jnp.dot(e, wee_ref[...], preferred_element_type=jnp.float32, precision=lax.Precision.HIGHEST)
    z = z + deg * be_ref[...] + h_ref[...]
    hid = jnp.dot(z, w1_ref[...], preferred_element_type=jnp.float32) + b1_ref[...]
    hid = jnp.maximum(hid, 0.0)
    o = jnp.dot(hid, w2_ref[...], preferred_element_type=jnp.float32) + b2_ref[...]
    if not last:
        o = jnp.maximum(o, 0.0)
    o_ref[...] = o


def _tc_mlp(p_pair, a_pair, h, weh, wee, be, w1, b1, w2, b2, last):
    nblk = _N // _MLP_B
    full = lambda i: (0, 0)
    return pl.pallas_call(
        functools.partial(_mlp_body, last),
        grid=(nblk,),
        in_specs=[
            pl.BlockSpec((_NC, _MLP_B, _EMB), lambda i: (0, i, 0)),
            pl.BlockSpec((_NC, _MLP_B, _EMB), lambda i: (0, i, 0)),
            pl.BlockSpec((_MLP_B, _EMB), lambda i: (i, 0)),
            pl.BlockSpec((_EMB, _EMB), full),
            pl.BlockSpec((_DE, _EMB), full),
            pl.BlockSpec((1, _EMB), full),
            pl.BlockSpec((_EMB, 2 * _EMB), full),
            pl.BlockSpec((1, 2 * _EMB), full),
            pl.BlockSpec((2 * _EMB, _EMB), full),
            pl.BlockSpec((1, _EMB), full),
        ],
        out_specs=pl.BlockSpec((_MLP_B, _EMB), lambda i: (i, 0)),
        out_shape=jax.ShapeDtypeStruct((_N, _EMB), jnp.float32),
    )(p_pair, a_pair, h, weh, wee, be, w1, b1, w2, b2)


def _pool_body(h_ref, batch_ref, wg1_ref, bg1_ref, wg2_ref, bg2_ref, o_ref):
    b = batch_ref[...]
    g_iota = lax.broadcasted_iota(jnp.int32, (_G, _N), 0)
    n_iota = lax.broadcasted_iota(jnp.int32, (_G, _N), 1)
    eq = b == g_iota
    lastidx = jnp.max(jnp.where(eq, n_iota, -1), axis=1, keepdims=True)
    present = lastidx >= 0
    pres_row = jnp.where(present, 1.0, 0.0).reshape(1, _G)
    r_iota = lax.broadcasted_iota(jnp.int32, (_G, _G), 0)
    c_iota = lax.broadcasted_iota(jnp.int32, (_G, _G), 1)
    tri = jnp.where(r_iota < c_iota, 1.0, 0.0)
    ranks = jnp.dot(pres_row, tri, preferred_element_type=jnp.float32, precision=lax.Precision.HIGHEST).astype(jnp.int32)
    nb = jnp.sum(pres_row)
    sel = jnp.where(eq & (n_iota == lastidx), 1.0, 0.0)
    m = jnp.dot(sel, h_ref[...], preferred_element_type=jnp.float32, precision=lax.Precision.HIGHEST)
    k_iota = lax.broadcasted_iota(jnp.int32, (_G, _G), 0)
    reord = jnp.where((ranks == k_iota) & (pres_row > 0), 1.0, 0.0)
    sup = jnp.dot(reord, m, preferred_element_type=jnp.float32, precision=lax.Precision.HIGHEST)
    krow = lax.broadcasted_iota(jnp.int32, (_G, 1), 0)
    sup = jnp.where(krow >= nb.astype(jnp.int32), h_ref[0:1, :], sup)
    v = jnp.dot(sup, wg1_ref[...], preferred_element_type=jnp.float32) + bg1_ref[...]
    v = jnp.where(v > 0, v, jnp.exp(jnp.minimum(v, 0.0)) - 1.0)
    o_ref[...] = jnp.dot(v, wg2_ref[...], preferred_element_type=jnp.float32) + bg2_ref[...]


def _tc_pool(h, batch_row, wg1, bg1, wg2, bg2):
    return pl.pallas_call(
        _pool_body,
        out_shape=jax.ShapeDtypeStruct((_G, _TASKS), jnp.float32),
    )(h, batch_row, wg1, bg1, wg2, bg2)


def kernel(x, edge_index, edge_attr, batch, W_edge, b_edge, W1, b1, W2, b2,
           Wg1, bg1, Wg2, bg2):
    src = edge_index[0]
    dst = edge_index[1]
    dst3 = dst.reshape(_NW, _NCH, _CH)
    zeros_big = jnp.zeros((_N, _EMB), jnp.float32)
    ea_aug = jnp.concatenate(
        [edge_attr, jnp.ones((_E, 1), jnp.float32),
         jnp.zeros((_E, _EMB - _DE - 1), jnp.float32)], axis=1)

    a_pair = _sc_edge_sums(ea_aug, dst3, zeros_big)

    h = x
    for l in range(_NUM_LAYER):
        p_pair = _sc_gather_scatter(h, src, dst, zeros_big)
        h = _tc_mlp(p_pair, a_pair, h,
                    W_edge[l, :_EMB], W_edge[l, _EMB:],
                    b_edge[l].reshape(1, _EMB),
                    W1[l], b1[l].reshape(1, 2 * _EMB),
                    W2[l], b2[l].reshape(1, _EMB),
                    last=(l == _NUM_LAYER - 1))

    return _tc_pool(h, batch.reshape(1, _N), Wg1, bg1.reshape(1, _EMB // 2),
                    Wg2, bg2.reshape(1, _TASKS))


# speedup vs baseline: 1.1439x; 1.1439x over previous
import functools

import jax
import jax.numpy as jnp
from jax import lax
from jax.experimental import pallas as pl
from jax.experimental.pallas import tpu as pltpu
from jax.experimental.pallas import tpu_sc as plsc

_NUM_LAYER = 3
_EMB = 128
_DE = 16
_N = 10000
_E = 320000
_TASKS = 10
_G = 128

_NC = 2
_NS = 16
_NW = _NC * _NS
_EPT = _E // _NW
_CH = 80
_NCH = _EPT // _CH
_RA = 632
_RT0 = _RA * (_NS - 1)
_RT = _N - _RT0

_MLP_B = 1000


def _sc_mesh():
    return plsc.VectorSubcoreMesh(
        core_axis_name="c", subcore_axis_name="s",
        num_cores=_NC, num_subcores=_NS)


_NBUF = 3


def _sc_gather_scatter_body(h_hbm, src_hbm, dst3_hbm, zero_hbm, out_hbm,
                            srcb0, srcb1, srcb2, dst_all, rows0, rows1, rows2,
                            acc_sh, sem0, sem1, sem2):
    c = lax.axis_index("c")
    s = lax.axis_index("s")
    srcb = [srcb0, srcb1, srcb2]
    rows = [rows0, rows1, rows2]
    sems = [sem0, sem1, sem2]
    wid = c * _NS + s

    @pl.when(s < _NS - 1)
    def _():
        pltpu.sync_copy(zero_hbm.at[pl.ds(s * _RA, _RA)],
                        acc_sh.at[pl.ds(s * _RA, _RA)])

    @pl.when(s == _NS - 1)
    def _():
        pltpu.sync_copy(zero_hbm.at[pl.ds(_RT0, _RT)],
                        acc_sh.at[pl.ds(_RT0, _RT)])

    pltpu.sync_copy(dst3_hbm.at[wid], dst_all)
    plsc.subcore_barrier()

    base = wid * _EPT

    def gather(i, j):
        pltpu.sync_copy(src_hbm.at[pl.ds(base + i * _CH, _CH)], srcb[j])
        pltpu.async_copy(h_hbm.at[srcb[j]], rows[j], sems[j])

    def drain(i, j):
        pltpu.make_async_copy(h_hbm.at[srcb[j]], rows[j], sems[j]).wait()
        pltpu.sync_copy(rows[j], acc_sh.at[dst_all.at[i]], add=True)

    for j in range(_NBUF - 1):
        gather(j, j)

    def group(q, carry):
        i0 = q * _NBUF
        for j in range(_NBUF):
            drain(i0 + j, j)
            gather(i0 + j + (_NBUF - 1), (j + _NBUF - 1) % _NBUF)
        return carry

    nq = (_NCH - _NBUF + 1) // _NBUF
    lax.fori_loop(0, nq, group, 0)

    for i in range(nq * _NBUF, _NCH):
        drain(i, i % _NBUF)
        nxt = i + _NBUF - 1
        if nxt < _NCH:
            gather(nxt, nxt % _NBUF)

    plsc.subcore_barrier()

    @pl.when(s < _NS - 1)
    def _():
        pltpu.sync_copy(acc_sh.at[pl.ds(s * _RA, _RA)],
                        out_hbm.at[c, pl.ds(s * _RA, _RA)])

    @pl.when(s == _NS - 1)
    def _():
        pltpu.sync_copy(acc_sh.at[pl.ds(_RT0, _RT)],
                        out_hbm.at[c, pl.ds(_RT0, _RT)])


@functools.lru_cache(maxsize=None)
def _sc_gather_scatter_kernel():
    return pl.kernel(
        _sc_gather_scatter_body,
        out_type=jax.ShapeDtypeStruct((_NC, _N, _EMB), jnp.float32),
        mesh=_sc_mesh(),
        scratch_types=[
            pltpu.VMEM((_CH,), jnp.int32),
            pltpu.VMEM((_CH,), jnp.int32),
            pltpu.VMEM((_CH,), jnp.int32),
            pltpu.VMEM((_NCH, _CH), jnp.int32),
            pltpu.VMEM((_CH, _EMB), jnp.float32),
            pltpu.VMEM((_CH, _EMB), jnp.float32),
            pltpu.VMEM((_CH, _EMB), jnp.float32),
            pltpu.VMEM_SHARED((_N, _EMB), jnp.float32),
            pltpu.SemaphoreType.DMA,
            pltpu.SemaphoreType.DMA,
            pltpu.SemaphoreType.DMA,
        ],
    )


def _sc_gather_scatter(h, src, dst3, zeros_big):
    return _sc_gather_scatter_kernel()(h, src, dst3, zeros_big)


def _sc_linear_scatter_body(rows_hbm, dst3_hbm, zero_hbm, out_hbm,
                            dst_all, rows0, rows1, rows2,
                            acc_sh, sem0, sem1, sem2):
    c = lax.axis_index("c")
    s = lax.axis_index("s")
    rows = [rows0, rows1, rows2]
    sems = [sem0, sem1, sem2]
    wid = c * _NS + s

    @pl.when(s < _NS - 1)
    def _():
        pltpu.sync_copy(zero_hbm.at[pl.ds(s * _RA, _RA)],
                        acc_sh.at[pl.ds(s * _RA, _RA)])

    @pl.when(s == _NS - 1)
    def _():
        pltpu.sync_copy(zero_hbm.at[pl.ds(_RT0, _RT)],
                        acc_sh.at[pl.ds(_RT0, _RT)])

    pltpu.sync_copy(dst3_hbm.at[wid], dst_all)
    plsc.subcore_barrier()

    base = wid * _EPT

    def gather(i, j):
        pltpu.async_copy(rows_hbm.at[pl.ds(base + i * _CH, _CH)],
                         rows[j], sems[j])

    def drain(i, j):
        pltpu.make_async_copy(rows_hbm.at[pl.ds(base, _CH)],
                              rows[j], sems[j]).wait()
        pltpu.sync_copy(rows[j], acc_sh.at[dst_all.at[i]], add=True)

    for j in range(_NBUF - 1):
        gather(j, j)

    def group(q, carry):
        i0 = q * _NBUF
        for j in range(_NBUF):
            drain(i0 + j, j)
            gather(i0 + j + (_NBUF - 1), (j + _NBUF - 1) % _NBUF)
        return carry

    nq = (_NCH - _NBUF + 1) // _NBUF
    lax.fori_loop(0, nq, group, 0)

    for i in range(nq * _NBUF, _NCH):
        drain(i, i % _NBUF)
        nxt = i + _NBUF - 1
        if nxt < _NCH:
            gather(nxt, nxt % _NBUF)

    plsc.subcore_barrier()

    @pl.when(s < _NS - 1)
    def _():
        pltpu.sync_copy(acc_sh.at[pl.ds(s * _RA, _RA)],
                        out_hbm.at[c, pl.ds(s * _RA, _RA)])

    @pl.when(s == _NS - 1)
    def _():
        pltpu.sync_copy(acc_sh.at[pl.ds(_RT0, _RT)],
                        out_hbm.at[c, pl.ds(_RT0, _RT)])


@functools.lru_cache(maxsize=None)
def _sc_linear_scatter_kernel():
    return pl.kernel(
        _sc_linear_scatter_body,
        out_type=jax.ShapeDtypeStruct((_NC, _N, _EMB), jnp.float32),
        mesh=_sc_mesh(),
        scratch_types=[
            pltpu.VMEM((_NCH, _CH), jnp.int32),
            pltpu.VMEM((_CH, _EMB), jnp.float32),
            pltpu.VMEM((_CH, _EMB), jnp.float32),
            pltpu.VMEM((_CH, _EMB), jnp.float32),
            pltpu.VMEM_SHARED((_N, _EMB), jnp.float32),
            pltpu.SemaphoreType.DMA,
            pltpu.SemaphoreType.DMA,
            pltpu.SemaphoreType.DMA,
        ],
    )


def _sc_edge_sums(ea_aug, dst3, zeros_big):
    return _sc_linear_scatter_kernel()(ea_aug, dst3, zeros_big)


def _mlp_body(last, p_ref, a_ref, h_ref, weh_ref, wee_ref, be_ref,
              w1_ref, b1_ref, w2_ref, b2_ref, o_ref):
    p = p_ref[0] + p_ref[1]
    a = a_ref[0] + a_ref[1]
    e = a[:, :_DE]
    deg = a[:, _DE:_DE + 1]
    z = jnp.dot(p, weh_ref[...], preferred_element_type=jnp.float32, precision=lax.Precision.HIGHEST)
    z = z + jnp.dot(e, wee_ref[...], preferred_element_type=jnp.float32, precision=lax.Precision.HIGHEST)
    z = z + deg * be_ref[...] + h_ref[...]
    hid = jnp.dot(z, w1_ref[...], preferred_element_type=jnp.float32) + b1_ref[...]
    hid = jnp.maximum(hid, 0.0)
    o = jnp.dot(hid, w2_ref[...], preferred_element_type=jnp.float32) + b2_ref[...]
    if not last:
        o = jnp.maximum(o, 0.0)
    o_ref[...] = o


def _tc_mlp(p_pair, a_pair, h, weh, wee, be, w1, b1, w2, b2, last):
    nblk = _N // _MLP_B
    full = lambda i: (0, 0)
    return pl.pallas_call(
        functools.partial(_mlp_body, last),
        grid=(nblk,),
        in_specs=[
            pl.BlockSpec((_NC, _MLP_B, _EMB), lambda i: (0, i, 0)),
            pl.BlockSpec((_NC, _MLP_B, _EMB), lambda i: (0, i, 0)),
            pl.BlockSpec((_MLP_B, _EMB), lambda i: (i, 0)),
            pl.BlockSpec((_EMB, _EMB), full),
            pl.BlockSpec((_DE, _EMB), full),
            pl.BlockSpec((1, _EMB), full),
            pl.BlockSpec((_EMB, 2 * _EMB), full),
            pl.BlockSpec((1, 2 * _EMB), full),
            pl.BlockSpec((2 * _EMB, _EMB), full),
            pl.BlockSpec((1, _EMB), full),
        ],
        out_specs=pl.BlockSpec((_MLP_B, _EMB), lambda i: (i, 0)),
        out_shape=jax.ShapeDtypeStruct((_N, _EMB), jnp.float32),
    )(p_pair, a_pair, h, weh, wee, be, w1, b1, w2, b2)


def _pool_body(h_ref, batch_ref, wg1_ref, bg1_ref, wg2_ref, bg2_ref, o_ref):
    b = batch_ref[...]
    g_iota = lax.broadcasted_iota(jnp.int32, (_G, _N), 0)
    n_iota = lax.broadcasted_iota(jnp.int32, (_G, _N), 1)
    eq = b == g_iota
    lastidx = jnp.max(jnp.where(eq, n_iota, -1), axis=1, keepdims=True)
    present = lastidx >= 0
    pres_row = jnp.where(present, 1.0, 0.0).reshape(1, _G)
    r_iota = lax.broadcasted_iota(jnp.int32, (_G, _G), 0)
    c_iota = lax.broadcasted_iota(jnp.int32, (_G, _G), 1)
    tri = jnp.where(r_iota < c_iota, 1.0, 0.0)
    ranks = jnp.dot(pres_row, tri, preferred_element_type=jnp.float32, precision=lax.Precision.HIGHEST).astype(jnp.int32)
    nb = jnp.sum(pres_row)
    sel = jnp.where(eq & (n_iota == lastidx), 1.0, 0.0)
    m = jnp.dot(sel, h_ref[...], preferred_element_type=jnp.float32, precision=lax.Precision.HIGHEST)
    k_iota = lax.broadcasted_iota(jnp.int32, (_G, _G), 0)
    reord = jnp.where((ranks == k_iota) & (pres_row > 0), 1.0, 0.0)
    sup = jnp.dot(reord, m, preferred_element_type=jnp.float32, precision=lax.Precision.HIGHEST)
    krow = lax.broadcasted_iota(jnp.int32, (_G, 1), 0)
    sup = jnp.where(krow >= nb.astype(jnp.int32), h_ref[0:1, :], sup)
    v = jnp.dot(sup, wg1_ref[...], preferred_element_type=jnp.float32) + bg1_ref[...]
    v = jnp.where(v > 0, v, jnp.exp(jnp.minimum(v, 0.0)) - 1.0)
    o_ref[...] = jnp.dot(v, wg2_ref[...], preferred_element_type=jnp.float32) + bg2_ref[...]


def _tc_pool(h, batch_row, wg1, bg1, wg2, bg2):
    return pl.pallas_call(
        _pool_body,
        out_shape=jax.ShapeDtypeStruct((_G, _TASKS), jnp.float32),
    )(h, batch_row, wg1, bg1, wg2, bg2)


def kernel(x, edge_index, edge_attr, batch, W_edge, b_edge, W1, b1, W2, b2,
           Wg1, bg1, Wg2, bg2):
    src = edge_index[0]
    dst = edge_index[1]
    dst3 = dst.reshape(_NW, _NCH, _CH)
    zeros_big = jnp.zeros((_N, _EMB), jnp.float32)
    ea_aug = jnp.concatenate(
        [edge_attr, jnp.ones((_E, 1), jnp.float32),
         jnp.zeros((_E, _EMB - _DE - 1), jnp.float32)], axis=1)

    a_pair = _sc_edge_sums(ea_aug, dst3, zeros_big)

    h = x
    for l in range(_NUM_LAYER):
        p_pair = _sc_gather_scatter(h, src, dst3, zeros_big)
        h = _tc_mlp(p_pair, a_pair, h,
                    W_edge[l, :_EMB], W_edge[l, _EMB:],
                    b_edge[l].reshape(1, _EMB),
                    W1[l], b1[l].reshape(1, 2 * _EMB),
                    W2[l], b2[l].reshape(1, _EMB),
                    last=(l == _NUM_LAYER - 1))

    return _tc_pool(h, batch.reshape(1, _N), Wg1, bg1.reshape(1, _EMB // 2),
                    Wg2, bg2.reshape(1, _TASKS))
